# baseline (speedup 1.0000x reference)
import functools

import jax
import jax.numpy as jnp
from jax import lax
from jax.experimental import pallas as pl
from jax.experimental.pallas import tpu as pltpu
from jax.experimental.pallas import tpu_sc as plsc

VOCAB = 100000
EMBED_DIM = 64
BATCH = 4096
HIST = 50

NC, NS = 2, 16
NW = NC * NS
BW = BATCH // NW
DT = EMBED_DIM // 8
ROWS_BYTES = BW * EMBED_DIM * 4
TROWS_BYTES = EMBED_DIM * BW * 4


@functools.partial(
    pl.kernel,
    mesh=plsc.VectorSubcoreMesh(core_axis_name="c", subcore_axis_name="s"),
    compiler_params=pltpu.CompilerParams(use_tc_tiling_on_sc=False,
                                         needs_layout_passes=False),
    out_type=jax.ShapeDtypeStruct((HIST, DT, NW, 8, BW), jnp.float32),
    scratch_types=[
        pltpu.VMEM((HIST, BW), jnp.int32),
        pltpu.VMEM((BW, EMBED_DIM), jnp.float32),
        pltpu.VMEM((BW, EMBED_DIM), jnp.float32),
        pltpu.VMEM((DT, 8, BW), jnp.float32),
        pltpu.VMEM((DT, 8, BW), jnp.float32),
        pltpu.SemaphoreType.DMA,
        pltpu.SemaphoreType.DMA,
        pltpu.SemaphoreType.DMA,
        pltpu.SemaphoreType.DMA,
    ],
)
def _emb_gather(idx_hbm, table_hbm, out_hbm, idx_v, rows_a, rows_b,
                trows_a, trows_b, gsem_a, gsem_b, osem_a, osem_b):
    wid = lax.axis_index("s") * NC + lax.axis_index("c")
    rows = (rows_a, rows_b)
    trows = (trows_a, trows_b)
    gsem = (gsem_a, gsem_b)
    osem = (osem_a, osem_b)

    pltpu.sync_copy(idx_hbm.at[pl.ds(0, HIST), pl.ds(wid * BW, BW)], idx_v)
    pltpu.async_copy(table_hbm.at[idx_v.at[0]], rows[0], gsem[0])

    iota = lax.iota(jnp.int32, 16)
    bidx_list = [iota + bq * 16 for bq in range(BW // 16)]

    def transpose_block(p):
        @plsc.parallel_loop(0, 16, unroll=4)
        def _(k):
            perm = (iota + k) & 15
            for dq in range(EMBED_DIM // 16):
                dvec = perm + dq * 16
                dhi = dvec >> 3
                dlo = dvec & 7
                for bq in range(BW // 16):
                    bidx = bidx_list[bq]
                    vals = plsc.load_gather(rows[p], [bidx, dvec])
                    plsc.store_scatter(trows[p], [dhi, dlo, bidx], vals)

    def drain_gather(p):
        pltpu.make_async_copy(table_hbm.at[pl.ds(0, BW)], rows[p],
                              gsem[p]).wait()

    def trows_data(p):
        return trows[p].at[pl.ds(0, DT), pl.ds(0, 8), pl.ds(0, BW)]

    def drain_outcopy(p):
        pltpu.make_async_copy(trows_data(p), out_hbm.at[0, pl.ds(0, DT), 0],
                              osem[p]).wait()

    def body(g, carry):
        for p in (0, 1):
            h = 2 * g + p
            if p == 0:
                pltpu.async_copy(
                    table_hbm.at[idx_v.at[h + 1]], rows[1], gsem[1])
            else:
                @pl.when(g < (HIST // 2) - 1)
                def _():
                    pltpu.async_copy(
                        table_hbm.at[idx_v.at[h + 1]], rows[0], gsem[0])

            drain_gather(p)

            @pl.when(g >= 1)
            def _():
                drain_outcopy(p)

            transpose_block(p)

            pltpu.async_copy(trows_data(p), out_hbm.at[h, pl.ds(0, DT), wid],
                             osem[p])
        return carry

    lax.fori_loop(0, HIST // 2, body, 0)
    drain_outcopy(0)
    drain_outcopy(1)


def kernel(indices, table):
    o5 = _emb_gather(indices.astype(jnp.int32).T, table)
    return jnp.transpose(o5, (2, 4, 0, 1, 3)).reshape(BATCH, HIST, EMBED_DIM)


# speedup vs baseline: 1.0083x; 1.0083x over previous
import functools

import jax
import jax.numpy as jnp
from jax import lax
from jax.experimental import pallas as pl
from jax.experimental.pallas import tpu as pltpu
from jax.experimental.pallas import tpu_sc as plsc

VOCAB = 100000
EMBED_DIM = 64
BATCH = 4096
HIST = 50

NC, NS = 2, 16
NW = NC * NS
BW = BATCH // NW
DT = EMBED_DIM // 8
ROWS_BYTES = BW * EMBED_DIM * 4
TROWS_BYTES = EMBED_DIM * BW * 4


@functools.partial(
    pl.kernel,
    mesh=plsc.VectorSubcoreMesh(core_axis_name="c", subcore_axis_name="s"),
    compiler_params=pltpu.CompilerParams(use_tc_tiling_on_sc=False,
                                         needs_layout_passes=False),
    out_type=jax.ShapeDtypeStruct((HIST, DT, NW, 8, BW), jnp.float32),
    scratch_types=[
        pltpu.VMEM((HIST, BW), jnp.int32),
        pltpu.VMEM((BW, EMBED_DIM), jnp.float32),
        pltpu.VMEM((BW, EMBED_DIM), jnp.float32),
        pltpu.VMEM((DT, 8, BW), jnp.float32),
        pltpu.VMEM((DT, 8, BW), jnp.float32),
        pltpu.SemaphoreType.DMA,
        pltpu.SemaphoreType.DMA,
        pltpu.SemaphoreType.DMA,
        pltpu.SemaphoreType.DMA,
    ],
)
def _emb_gather(idx_hbm, table_hbm, out_hbm, idx_v, rows_a, rows_b,
                trows_a, trows_b, gsem_a, gsem_b, osem_a, osem_b):
    wid = lax.axis_index("s") * NC + lax.axis_index("c")
    rows = (rows_a, rows_b)
    trows = (trows_a, trows_b)
    gsem = (gsem_a, gsem_b)
    osem = (osem_a, osem_b)

    pltpu.sync_copy(idx_hbm.at[pl.ds(0, HIST), pl.ds(wid * BW, BW)], idx_v)
    pltpu.async_copy(table_hbm.at[idx_v.at[0]], rows[0], gsem[0])

    iota = lax.iota(jnp.int32, 16)
    bidx_list = [iota + bq * 16 for bq in range(BW // 16)]

    def transpose_dq(p, dq):
        @plsc.parallel_loop(0, 16, unroll=2)
        def _(k):
            perm = (iota + k) & 15
            dvec = perm + dq * 16
            dhi = dvec >> 3
            dlo = dvec & 7
            for bq in range(BW // 16):
                bidx = bidx_list[bq]
                vals = plsc.load_gather(rows[p], [bidx, dvec])
                plsc.store_scatter(trows[p], [dhi, dlo, bidx], vals)

    def drain_gather(p):
        pltpu.make_async_copy(table_hbm.at[pl.ds(0, BW)], rows[p],
                              gsem[p]).wait()

    def trows_data(p):
        return trows[p].at[pl.ds(0, DT), pl.ds(0, 8), pl.ds(0, BW)]

    def drain_outcopy(p):
        pltpu.make_async_copy(trows_data(p), out_hbm.at[0, pl.ds(0, DT), 0],
                              osem[p]).wait()

    def body(g, carry):
        for p in (0, 1):
            h = 2 * g + p
            if p == 0:
                pltpu.async_copy(
                    table_hbm.at[idx_v.at[h + 1]], rows[1], gsem[1])
            else:
                @pl.when(g < (HIST // 2) - 1)
                def _():
                    pltpu.async_copy(
                        table_hbm.at[idx_v.at[h + 1]], rows[0], gsem[0])

            drain_gather(p)

            @pl.when(g >= 1)
            def _():
                drain_outcopy(p)

            for dq in range(EMBED_DIM // 16):
                transpose_dq(p, dq)
                pltpu.async_copy(
                    trows[p].at[pl.ds(dq * 2, 2)],
                    out_hbm.at[h, pl.ds(dq * 2, 2), wid], osem[p])
        return carry

    lax.fori_loop(0, HIST // 2, body, 0)
    drain_outcopy(0)
    drain_outcopy(1)


def kernel(indices, table):
    o5 = _emb_gather(indices.astype(jnp.int32).T, table)
    return jnp.transpose(o5, (2, 4, 0, 1, 3)).reshape(BATCH, HIST, EMBED_DIM)


# speedup vs baseline: 1.0106x; 1.0023x over previous
import functools

import jax
import jax.numpy as jnp
from jax import lax
from jax.experimental import pallas as pl
from jax.experimental.pallas import tpu as pltpu
from jax.experimental.pallas import tpu_sc as plsc

VOCAB = 100000
EMBED_DIM = 64
BATCH = 4096
HIST = 50

NC, NS = 2, 16
NW = NC * NS
BW = BATCH // NW
DT = EMBED_DIM // 8


@functools.partial(
    pl.kernel,
    mesh=plsc.VectorSubcoreMesh(core_axis_name="c", subcore_axis_name="s"),
    compiler_params=pltpu.CompilerParams(use_tc_tiling_on_sc=False,
                                         needs_layout_passes=False),
    out_type=jax.ShapeDtypeStruct((HIST, DT, NW, 8, BW), jnp.float32),
    scratch_types=[
        pltpu.VMEM((HIST, BW), jnp.int32),
        pltpu.VMEM((BW, EMBED_DIM), jnp.float32),
        pltpu.VMEM((BW, EMBED_DIM), jnp.float32),
        pltpu.VMEM((DT, 8, BW), jnp.float32),
        pltpu.VMEM((DT, 8, BW), jnp.float32),
        pltpu.SemaphoreType.DMA,
        pltpu.SemaphoreType.DMA,
        pltpu.SemaphoreType.DMA,
        pltpu.SemaphoreType.DMA,
    ],
)
def _emb_gather(idx_hbm, table_hbm, out_hbm, idx_v, rows_a, rows_b,
                trows_a, trows_b, gsem_a, gsem_b, osem_a, osem_b):
    wid = lax.axis_index("s") * NC + lax.axis_index("c")
    rows = (rows_a, rows_b)
    trows = (trows_a, trows_b)
    gsem = (gsem_a, gsem_b)
    osem = (osem_a, osem_b)

    pltpu.sync_copy(idx_hbm.at[pl.ds(0, HIST), pl.ds(wid * BW, BW)], idx_v)
    pltpu.async_copy(table_hbm.at[idx_v.at[0]], rows[0], gsem[0])

    iota = lax.iota(jnp.int32, 16)
    bidx_list = [iota + bq * 16 for bq in range(BW // 16)]

    def transpose_block(p):
        @plsc.parallel_loop(0, 16, unroll=2)
        def _(k):
            perm = (iota + k) & 15
            for dq in range(EMBED_DIM // 16):
                dvec = perm + dq * 16
                dhi = dvec >> 3
                dlo = dvec & 7
                for bq in range(BW // 16):
                    bidx = bidx_list[bq]
                    vals = plsc.load_gather(rows[p], [bidx, dvec])
                    plsc.store_scatter(trows[p], [dhi, dlo, bidx], vals)

    def drain_gather(p):
        pltpu.make_async_copy(table_hbm.at[pl.ds(0, BW)], rows[p],
                              gsem[p]).wait()

    def trows_data(p):
        return trows[p].at[pl.ds(0, DT), pl.ds(0, 8), pl.ds(0, BW)]

    def drain_outcopy(p):
        pltpu.make_async_copy(trows_data(p), out_hbm.at[0, pl.ds(0, DT), 0],
                              osem[p]).wait()

    def body(g, carry):
        for p in (0, 1):
            h = 2 * g + p
            if p == 0:
                pltpu.async_copy(
                    table_hbm.at[idx_v.at[h + 1]], rows[1], gsem[1])
            else:
                @pl.when(g < (HIST // 2) - 1)
                def _():
                    pltpu.async_copy(
                        table_hbm.at[idx_v.at[h + 1]], rows[0], gsem[0])

            drain_gather(p)

            @pl.when(g >= 1)
            def _():
                drain_outcopy(p)

            transpose_block(p)
            pltpu.async_copy(trows_data(p), out_hbm.at[h, pl.ds(0, DT), wid],
                             osem[p])
        return carry

    lax.fori_loop(0, HIST // 2, body, 0)
    drain_outcopy(0)
    drain_outcopy(1)


def kernel(indices, table):
    o5 = _emb_gather(indices.astype(jnp.int32).T, table)
    return jnp.transpose(o5, (2, 4, 0, 1, 3)).reshape(BATCH, HIST, EMBED_DIM)
